# Initial kernel scaffold; baseline (speedup 1.0000x reference)
import jax
import jax.numpy as jnp
from jax.experimental import pallas as pl


def kernel(x, lengths, emb, w1, b1, w2, b2):
    raise NotImplementedError("write your pallas kernel here")


# speedup vs baseline: 81.9203x; 81.9203x over previous
import functools

import jax
import jax.numpy as jnp
from jax import lax
from jax.experimental import pallas as pl
from jax.experimental.pallas import tpu as pltpu
from jax.experimental.pallas import tpu_sc as plsc

_NC, _NS, _LANES = 2, 16, 16
_NW = _NC * _NS


def _build_sc_histogram(B, L, vocab, chunk):
    rows_per_w = B // _NW
    n_chunks = rows_per_w // chunk
    groups = chunk // _LANES
    mesh = plsc.VectorSubcoreMesh(
        core_axis_name="c", subcore_axis_name="s",
        num_cores=_NC, num_subcores=_NS)

    @functools.partial(
        pl.kernel,
        out_type=jax.ShapeDtypeStruct((B, vocab), jnp.float32),
        mesh=mesh,
        compiler_params=pltpu.CompilerParams(
            needs_layout_passes=False, use_tc_tiling_on_sc=False),
        scratch_types=[
            pltpu.VMEM((chunk * L,), jnp.int32),
            pltpu.VMEM((chunk,), jnp.int32),
            pltpu.VMEM((chunk, vocab), jnp.float32),
        ],
    )
    def sc_histogram(x_hbm, len_hbm, counts_hbm, x_v, len_v, counts_v):
        wid = lax.axis_index("s") * _NC + lax.axis_index("c")
        lane = lax.iota(jnp.int32, _LANES)
        ones = jnp.ones((_LANES,), jnp.float32)
        zeros = jnp.zeros((_LANES,), jnp.float32)
        base0 = wid * rows_per_w
        for ci in range(n_chunks):
            base = base0 + ci * chunk
            pltpu.sync_copy(x_hbm.at[pl.ds(base * L, chunk * L)], x_v)
            pltpu.sync_copy(len_hbm.at[pl.ds(base, chunk)], len_v)

            @pl.loop(0, chunk)
            def _(r):
                for cc in range(vocab // _LANES):
                    counts_v[r, pl.ds(cc * _LANES, _LANES)] = zeros

            rows = [g * _LANES + lane for g in range(groups)]
            fbase = [r * L for r in rows]
            lens = [len_v[pl.ds(g * _LANES, _LANES)] for g in range(groups)]

            @pl.loop(0, L)
            def _(j):
                for g in range(groups):
                    tok = plsc.load_gather(x_v, [fbase[g] + j])
                    plsc.addupdate_scatter(
                        counts_v, [rows[g], tok], ones, mask=lens[g] > j)

            pltpu.sync_copy(counts_v, counts_hbm.at[pl.ds(base, chunk)])

    return sc_histogram


def _mlp_body(counts_ref, len_ref, emb_ref, w1_ref, b1_ref, w2_ref, b2_ref,
              out_ref):
    hi = jax.lax.Precision.HIGHEST
    s = jnp.dot(counts_ref[...], emb_ref[...],
                preferred_element_type=jnp.float32, precision=hi)
    rep = s / (len_ref[...] + 1e-8)
    h = jnp.dot(rep, w1_ref[...],
                preferred_element_type=jnp.float32, precision=hi)
    h = jnp.maximum(h + b1_ref[...], 0.0)
    out = jnp.dot(h, w2_ref[...],
                  preferred_element_type=jnp.float32, precision=hi)
    out_ref[...] = out + b2_ref[...]


def kernel(x, lengths, emb, w1, b1, w2, b2):
    B, L = x.shape
    vocab, dim = emb.shape
    hid, out_d = w2.shape[0], w2.shape[1]

    x_flat = jnp.reshape(x.astype(jnp.int32), (B * L,))
    counts = _build_sc_histogram(B, L, vocab, chunk=128)(
        x_flat, lengths.astype(jnp.int32))

    lenf = lengths.astype(jnp.float32).reshape(B, 1)
    bt = 2048
    grid = (B // bt,)
    logits = pl.pallas_call(
        _mlp_body,
        grid=grid,
        in_specs=[
            pl.BlockSpec((bt, vocab), lambda i: (i, 0)),
            pl.BlockSpec((bt, 1), lambda i: (i, 0)),
            pl.BlockSpec((vocab, dim), lambda i: (0, 0)),
            pl.BlockSpec((dim, hid), lambda i: (0, 0)),
            pl.BlockSpec((1, hid), lambda i: (0, 0)),
            pl.BlockSpec((hid, out_d), lambda i: (0, 0)),
            pl.BlockSpec((1, out_d), lambda i: (0, 0)),
        ],
        out_specs=pl.BlockSpec((bt, out_d), lambda i: (i, 0)),
        out_shape=jax.ShapeDtypeStruct((B, out_d), jnp.float32),
    )(counts, lenf, emb, w1, b1.reshape(1, hid), w2, b2.reshape(1, out_d))
    return logits


# speedup vs baseline: 103.5941x; 1.2646x over previous
import functools

import jax
import jax.numpy as jnp
from jax import lax
from jax.experimental import pallas as pl
from jax.experimental.pallas import tpu as pltpu
from jax.experimental.pallas import tpu_sc as plsc

_NC, _NS, _LANES = 2, 16, 16
_NW = _NC * _NS


def _build_sc_histogram(B, L, vocab, chunk):
    rows_per_w = B // _NW
    n_chunks = rows_per_w // chunk
    groups = chunk // _LANES
    mesh = plsc.VectorSubcoreMesh(
        core_axis_name="c", subcore_axis_name="s",
        num_cores=_NC, num_subcores=_NS)

    @functools.partial(
        pl.kernel,
        out_type=jax.ShapeDtypeStruct((B, vocab), jnp.float32),
        mesh=mesh,
        compiler_params=pltpu.CompilerParams(
            needs_layout_passes=False, use_tc_tiling_on_sc=False),
        scratch_types=[
            pltpu.VMEM((chunk * L,), jnp.int32),
            pltpu.VMEM((chunk,), jnp.int32),
            pltpu.VMEM((chunk, vocab), jnp.float32),
        ],
    )
    def sc_histogram(x_hbm, len_hbm, counts_hbm, x_v, len_v, counts_v):
        wid = lax.axis_index("s") * _NC + lax.axis_index("c")
        lane = lax.iota(jnp.int32, _LANES)
        ones = jnp.ones((_LANES,), jnp.float32)
        zeros = jnp.zeros((_LANES,), jnp.float32)
        base0 = wid * rows_per_w
        for ci in range(n_chunks):
            base = base0 + ci * chunk
            pltpu.sync_copy(x_hbm.at[pl.ds(base * L, chunk * L)], x_v)
            pltpu.sync_copy(len_hbm.at[pl.ds(base, chunk)], len_v)

            @pl.loop(0, chunk, unroll=8)
            def _(r):
                for cc in range(vocab // _LANES):
                    counts_v[r, pl.ds(cc * _LANES, _LANES)] = zeros

            rows = [g * _LANES + lane for g in range(groups)]
            fbase = [r * L for r in rows]
            lens = [len_v[pl.ds(g * _LANES, _LANES)] for g in range(groups)]

            @pl.loop(0, L, unroll=2)
            def _(j):
                toks = [plsc.load_gather(x_v, [fbase[g] + j])
                        for g in range(groups)]
                masks = [lens[g] > j for g in range(groups)]
                for g in range(groups):
                    plsc.addupdate_scatter(
                        counts_v, [rows[g], toks[g]], ones, mask=masks[g])

            pltpu.sync_copy(counts_v, counts_hbm.at[pl.ds(base, chunk)])

    return sc_histogram


def _mlp_body(counts_ref, len_ref, emb_ref, w1_ref, b1_ref, w2_ref, b2_ref,
              out_ref):
    hi = jax.lax.Precision.HIGHEST
    s = jnp.dot(counts_ref[...], emb_ref[...],
                preferred_element_type=jnp.float32, precision=hi)
    rep = s / (len_ref[...] + 1e-8)
    h = jnp.dot(rep, w1_ref[...],
                preferred_element_type=jnp.float32, precision=hi)
    h = jnp.maximum(h + b1_ref[...], 0.0)
    out = jnp.dot(h, w2_ref[...],
                  preferred_element_type=jnp.float32, precision=hi)
    out_ref[...] = out + b2_ref[...]


def kernel(x, lengths, emb, w1, b1, w2, b2):
    B, L = x.shape
    vocab, dim = emb.shape
    hid, out_d = w2.shape[0], w2.shape[1]

    x_flat = jnp.reshape(x.astype(jnp.int32), (B * L,))
    counts = _build_sc_histogram(B, L, vocab, chunk=128)(
        x_flat, lengths.astype(jnp.int32))

    lenf = lengths.astype(jnp.float32).reshape(B, 1)
    bt = 2048
    grid = (B // bt,)
    logits = pl.pallas_call(
        _mlp_body,
        grid=grid,
        in_specs=[
            pl.BlockSpec((bt, vocab), lambda i: (i, 0)),
            pl.BlockSpec((bt, 1), lambda i: (i, 0)),
            pl.BlockSpec((vocab, dim), lambda i: (0, 0)),
            pl.BlockSpec((dim, hid), lambda i: (0, 0)),
            pl.BlockSpec((1, hid), lambda i: (0, 0)),
            pl.BlockSpec((hid, out_d), lambda i: (0, 0)),
            pl.BlockSpec((1, out_d), lambda i: (0, 0)),
        ],
        out_specs=pl.BlockSpec((bt, out_d), lambda i: (i, 0)),
        out_shape=jax.ShapeDtypeStruct((B, out_d), jnp.float32),
    )(counts, lenf, emb, w1, b1.reshape(1, hid), w2, b2.reshape(1, out_d))
    return logits
